# baseline (speedup 1.0000x reference)
import jax
import jax.numpy as jnp
from jax.experimental import pallas as pl
from jax.experimental.pallas import tpu as pltpu

_CHUNKS = (96, 416, 512, 512, 416, 96)
_CMAX = max(_CHUNKS)
_NB = 3


def _offsets():
    offs, o = [], 0
    for c in _CHUNKS:
        offs.append(o)
        o += c
    return tuple(offs)


_OFFS = _offsets()


def _body(x_hbm, e_hbm, o_hbm, xbuf, ebuf, obuf, semx, seme, semo):
    n = len(_CHUNKS)

    def load(i):
        s = i % _NB
        rows, off = _CHUNKS[i], _OFFS[i]
        pltpu.make_async_copy(
            x_hbm.at[:, pl.ds(off, rows), :], xbuf.at[s, :, :rows, :],
            semx.at[s]).start()
        pltpu.make_async_copy(
            e_hbm.at[pl.ds(off, rows), :], ebuf.at[s, :rows, :],
            seme.at[s]).start()

    def wait_load(i):
        s = i % _NB
        rows, off = _CHUNKS[i], _OFFS[i]
        pltpu.make_async_copy(
            x_hbm.at[:, pl.ds(off, rows), :], xbuf.at[s, :, :rows, :],
            semx.at[s]).wait()
        pltpu.make_async_copy(
            e_hbm.at[pl.ds(off, rows), :], ebuf.at[s, :rows, :],
            seme.at[s]).wait()

    def store(i):
        s = i % _NB
        rows, off = _CHUNKS[i], _OFFS[i]
        return pltpu.make_async_copy(
            obuf.at[s, :, :rows, :], o_hbm.at[:, pl.ds(off, rows), :],
            semo.at[s])

    for i in range(min(_NB, n)):
        load(i)
    for i in range(n):
        s = i % _NB
        rows = _CHUNKS[i]
        wait_load(i)
        if i >= _NB:
            store(i - _NB).wait()
        obuf[s, :, :rows, :] = xbuf[s, :, :rows, :] + ebuf[s, :rows, :]
        store(i).start()
        if i + _NB < n:
            load(i + _NB)
    for i in range(n - _NB, n):
        store(i).wait()


def kernel(x, emb_table):
    B, S, D = x.shape
    return pl.pallas_call(
        _body,
        in_specs=[
            pl.BlockSpec(memory_space=pltpu.HBM),
            pl.BlockSpec(memory_space=pltpu.HBM),
        ],
        out_specs=pl.BlockSpec(memory_space=pltpu.HBM),
        out_shape=jax.ShapeDtypeStruct((B, S, D), x.dtype),
        scratch_shapes=[
            pltpu.VMEM((_NB, B, _CMAX, D), jnp.float32),
            pltpu.VMEM((_NB, _CMAX, D), jnp.float32),
            pltpu.VMEM((_NB, B, _CMAX, D), jnp.float32),
            pltpu.SemaphoreType.DMA((_NB,)),
            pltpu.SemaphoreType.DMA((_NB,)),
            pltpu.SemaphoreType.DMA((_NB,)),
        ],
    )(x, emb_table)


# speedup vs baseline: 1.0094x; 1.0094x over previous
import jax
import jax.numpy as jnp
from jax.experimental import pallas as pl
from jax.experimental.pallas import tpu as pltpu

_CHUNKS = (32, 64, 128, 256, 512, 512, 416, 96, 32)
_CMAX = max(_CHUNKS)
_NB = 3


def _offsets():
    offs, o = [], 0
    for c in _CHUNKS:
        offs.append(o)
        o += c
    return tuple(offs)


_OFFS = _offsets()


def _body(x_hbm, e_hbm, o_hbm, xbuf, ebuf, obuf, semx, seme, semo):
    n = len(_CHUNKS)

    def load(i):
        s = i % _NB
        rows, off = _CHUNKS[i], _OFFS[i]
        pltpu.make_async_copy(
            x_hbm.at[:, pl.ds(off, rows), :], xbuf.at[s, :, :rows, :],
            semx.at[s]).start()
        pltpu.make_async_copy(
            e_hbm.at[pl.ds(off, rows), :], ebuf.at[s, :rows, :],
            seme.at[s]).start()

    def wait_load(i):
        s = i % _NB
        rows, off = _CHUNKS[i], _OFFS[i]
        pltpu.make_async_copy(
            x_hbm.at[:, pl.ds(off, rows), :], xbuf.at[s, :, :rows, :],
            semx.at[s]).wait()
        pltpu.make_async_copy(
            e_hbm.at[pl.ds(off, rows), :], ebuf.at[s, :rows, :],
            seme.at[s]).wait()

    def store(i):
        s = i % _NB
        rows, off = _CHUNKS[i], _OFFS[i]
        return pltpu.make_async_copy(
            obuf.at[s, :, :rows, :], o_hbm.at[:, pl.ds(off, rows), :],
            semo.at[s])

    for i in range(min(_NB, n)):
        load(i)
    for i in range(n):
        s = i % _NB
        rows = _CHUNKS[i]
        wait_load(i)
        if i >= _NB:
            store(i - _NB).wait()
        obuf[s, :, :rows, :] = xbuf[s, :, :rows, :] + ebuf[s, :rows, :]
        store(i).start()
        if i + _NB < n:
            load(i + _NB)
    for i in range(n - _NB, n):
        store(i).wait()


def kernel(x, emb_table):
    B, S, D = x.shape
    return pl.pallas_call(
        _body,
        in_specs=[
            pl.BlockSpec(memory_space=pltpu.HBM),
            pl.BlockSpec(memory_space=pltpu.HBM),
        ],
        out_specs=pl.BlockSpec(memory_space=pltpu.HBM),
        out_shape=jax.ShapeDtypeStruct((B, S, D), x.dtype),
        scratch_shapes=[
            pltpu.VMEM((_NB, B, _CMAX, D), jnp.float32),
            pltpu.VMEM((_NB, _CMAX, D), jnp.float32),
            pltpu.VMEM((_NB, B, _CMAX, D), jnp.float32),
            pltpu.SemaphoreType.DMA((_NB,)),
            pltpu.SemaphoreType.DMA((_NB,)),
            pltpu.SemaphoreType.DMA((_NB,)),
        ],
    )(x, emb_table)
